# baseline (device time: 44317 ns/iter reference)
import jax
import jax.numpy as jnp
from jax import lax
from jax.experimental import pallas as pl
from jax.experimental.pallas import tpu as pltpu

N_DEV = 8
B, SQ, HQ, DH = 2, 512, 8, 64
WINDOW = 128
D_MODEL = 768
NSLOT = B * HQ
QLO = SQ - WINDOW
NQ1 = SQ - QLO
NK1 = WINDOW
NCHUNK = 8
SPC = NSLOT // NCHUNK
CPB = NCHUNK // B
FW = SPC * DH
T1C = NSLOT * DH
T1W = T1C + 128

F_PARENT = {1: 0, 3: 0, 4: 0, 2: 1, 5: 1, 6: 2, 7: 4}
F_CHILDREN = {0: [1, 3, 4], 1: [2, 5], 2: [6], 3: [], 4: [7],
              5: [], 6: [], 7: []}
PARTNERS = {
    0: [1, 3, 4],
    1: [0, 2, 5],
    2: [1, 6],
    3: [0],
    4: [0, 7],
    5: [1],
    6: [2],
    7: [4],
}

_BF = jnp.bfloat16
_MESH = pl.DeviceIdType.MESH


def kernel(x, Wq, K_ext, V_ext, Wo):
    Kt = jnp.transpose(K_ext, (0, 2, 3, 1))
    Vt = jnp.transpose(V_ext, (0, 2, 1, 3))

    def body(x_ref, wq_ref, k_ref, v_ref, wo_ref, out_ref,
             fbuf, t1buf, recv_f, send_f, t1_send, t1_recv):

        my = lax.axis_index("i")
        barrier = pltpu.get_barrier_semaphore()

        def barrier_round():
            for r in range(N_DEV):
                @pl.when(my == r)
                def _():
                    for p in PARTNERS[r]:
                        pl.semaphore_signal(barrier, inc=1, device_id=(p,),
                                            device_id_type=_MESH)
                    pl.semaphore_wait(barrier, len(PARTNERS[r]))

        barrier_round()

        def send_chunk(c, children, sent):
            for ci, child in enumerate(children):
                rdma = pltpu.make_async_remote_copy(
                    src_ref=fbuf.at[c],
                    dst_ref=fbuf.at[c],
                    send_sem=send_f.at[ci, c],
                    recv_sem=recv_f.at[c],
                    device_id=(child,),
                    device_id_type=_MESH,
                )
                rdma.start()
                sent.append(rdma)

        def project_chunk(c, wo_bf):
            b = c // CPB
            rows = (c % CPB) * FW
            part = jnp.dot(fbuf[c], wo_bf[rows:rows + FW, :],
                           preferred_element_type=jnp.float32)
            if c % CPB == 0:
                out_ref[b, :, :] = part
            else:
                out_ref[b, :, :] = out_ref[b] + part

        @pl.when(my == 1)
        def _():
            wq_bf = wq_ref[...].astype(_BF)
            for b in range(B):
                q_sub = jnp.dot(x_ref[b, QLO:SQ, :].astype(_BF), wq_bf,
                                preferred_element_type=jnp.float32)
                for h in range(HQ):
                    i = b * HQ + h
                    q_bh = q_sub[:, h * DH:(h + 1) * DH].astype(_BF)
                    s_mat = jnp.dot(q_bh, k_ref[b, h, :, 0:NK1].astype(_BF),
                                    preferred_element_type=jnp.float32) * 0.125
                    qi = QLO + lax.broadcasted_iota(jnp.int32, (NQ1, NK1), 0)
                    kj = SQ + lax.broadcasted_iota(jnp.int32, (NQ1, NK1), 1)
                    s_mat = jnp.where(jnp.abs(qi - kj) <= WINDOW, s_mat, -1e9)
                    m = jnp.max(s_mat, axis=1, keepdims=True)
                    w = jnp.exp(s_mat - m)
                    ssum = jnp.sum(w, axis=1, keepdims=True)
                    ctx = jnp.dot(w.astype(_BF),
                                  v_ref[b, h, 0:NK1, :].astype(_BF),
                                  preferred_element_type=jnp.float32)
                    t1buf[:, i * DH:(i + 1) * DH] = ctx
                    t1buf[:, T1C + i:T1C + i + 1] = m
                    t1buf[:, T1C + 16 + i:T1C + 16 + i + 1] = ssum
            rdma = pltpu.make_async_remote_copy(
                src_ref=t1buf, dst_ref=t1buf,
                send_sem=t1_send, recv_sem=t1_recv,
                device_id=(0,), device_id_type=_MESH,
            )
            rdma.start()
            rdma.wait_send()

        @pl.when(my == 0)
        def _():
            wq_bf = wq_ref[...].astype(_BF)
            t1_wait = pltpu.make_async_remote_copy(
                src_ref=t1buf, dst_ref=t1buf,
                send_sem=t1_send, recv_sem=t1_recv,
                device_id=(1,), device_id_type=_MESH,
            )
            sent = []
            first = [True]
            for b in range(B):
                q_b = jnp.dot(x_ref[b].astype(_BF), wq_bf,
                              preferred_element_type=jnp.float32)
                if first[0]:
                    t1_wait.wait_recv()
                    first[0] = False
                for h in range(HQ):
                    i = b * HQ + h
                    q_bh = q_b[:, h * DH:(h + 1) * DH].astype(_BF)
                    s_mat = jnp.dot(q_bh, k_ref[b, h].astype(_BF),
                                    preferred_element_type=jnp.float32) * 0.125
                    qi = lax.broadcasted_iota(jnp.int32, (SQ, SQ), 0)
                    kj = lax.broadcasted_iota(jnp.int32, (SQ, SQ), 1)
                    s_mat = jnp.where(jnp.abs(qi - kj) <= WINDOW, s_mat, -1e9)
                    m = jnp.max(s_mat, axis=1, keepdims=True)
                    w = jnp.exp(s_mat - m)
                    ssum = jnp.sum(w, axis=1, keepdims=True)
                    ctx = jnp.dot(w.astype(_BF), v_ref[b, h].astype(_BF),
                                  preferred_element_type=jnp.float32)
                    c, half = i // SPC, (i % SPC) * DH
                    fbuf[c, 0:QLO, half:half + DH] = (
                        ctx[0:QLO] * (1.0 / ssum[0:QLO])).astype(_BF)
                    m1 = t1buf[:, T1C + i:T1C + i + 1]
                    s1 = t1buf[:, T1C + 16 + i:T1C + 16 + i + 1]
                    c1 = t1buf[:, i * DH:(i + 1) * DH]
                    m0 = m[QLO:SQ]
                    mn = jnp.maximum(m0, m1)
                    a0 = jnp.exp(m0 - mn)
                    a1 = jnp.exp(m1 - mn)
                    sb = ssum[QLO:SQ] * a0 + s1 * a1
                    cb = ctx[QLO:SQ] * a0 + c1 * a1
                    fbuf[c, QLO:SQ, half:half + DH] = (
                        cb * (1.0 / sb)).astype(_BF)
                    if i % SPC == SPC - 1:
                        send_chunk(c, F_CHILDREN[0], sent)
            wo_bf = wo_ref[...].astype(_BF)
            for c in range(NCHUNK):
                project_chunk(c, wo_bf)
            for rdma in sent:
                rdma.wait_send()

        for r in range(1, N_DEV):
            @pl.when(my == r)
            def _(r=r):
                wo_bf = wo_ref[...].astype(_BF)
                sent = []
                for c in range(NCHUNK):
                    recv = pltpu.make_async_remote_copy(
                        src_ref=fbuf.at[c],
                        dst_ref=fbuf.at[c],
                        send_sem=send_f.at[0, c],
                        recv_sem=recv_f.at[c],
                        device_id=(F_PARENT[r],),
                        device_id_type=_MESH,
                    )
                    recv.wait_recv()
                    if F_CHILDREN[r]:
                        send_chunk(c, F_CHILDREN[r], sent)
                    project_chunk(c, wo_bf)
                for rdma in sent:
                    rdma.wait_send()

        barrier_round()

    return pl.pallas_call(
        body,
        out_shape=jax.ShapeDtypeStruct((B, SQ, D_MODEL), jnp.float32),
        in_specs=[pl.BlockSpec(memory_space=pltpu.VMEM)] * 5,
        out_specs=pl.BlockSpec(memory_space=pltpu.VMEM),
        scratch_shapes=[
            pltpu.VMEM((NCHUNK, SQ, FW), _BF),
            pltpu.VMEM((NQ1, T1W), jnp.float32),
            pltpu.SemaphoreType.DMA((NCHUNK,)),
            pltpu.SemaphoreType.DMA((3, NCHUNK)),
            pltpu.SemaphoreType.DMA,
            pltpu.SemaphoreType.DMA,
        ],
        compiler_params=pltpu.CompilerParams(collective_id=0),
    )(x, Wq, Kt, Vt, Wo)


# device time: 36489 ns/iter; 1.2145x vs baseline; 1.2145x over previous
import jax
import jax.numpy as jnp
from jax import lax
from jax.experimental import pallas as pl
from jax.experimental.pallas import tpu as pltpu

N_DEV = 8
B, SQ, HQ, DH = 2, 512, 8, 64
WINDOW = 128
D_MODEL = 768
NSLOT = B * HQ
QLO = SQ - WINDOW
NQ1 = SQ - QLO
NK1 = WINDOW
NCHUNK = 8
SPC = NSLOT // NCHUNK
CPB = NCHUNK // B
FW = SPC * DH
T1C = HQ * DH
T1W = T1C + 128

F_PARENT = {1: 0, 3: 0, 4: 0, 2: 1, 5: 1, 6: 2, 7: 4}
F_CHILDREN = {0: [1, 3, 4], 1: [2, 5], 2: [6], 3: [], 4: [7],
              5: [], 6: [], 7: []}
PARTNERS = {
    0: [1, 3, 4],
    1: [0, 2, 5],
    2: [1, 6],
    3: [0],
    4: [0, 7],
    5: [1],
    6: [2],
    7: [4],
}

_BF = jnp.bfloat16
_MESH = pl.DeviceIdType.MESH


def _penalty(nq, nk, q0, k0):
    qi = q0 + lax.broadcasted_iota(jnp.int32, (nq, nk), 0)
    kj = k0 + lax.broadcasted_iota(jnp.int32, (nq, nk), 1)
    return jnp.where(jnp.abs(qi - kj) <= WINDOW, 0.0, -1e9)


def kernel(x, Wq, K_ext, V_ext, Wo):
    Kt = jnp.transpose(K_ext, (0, 2, 3, 1))
    Vt = jnp.transpose(V_ext, (0, 2, 1, 3))

    def body(x_ref, wq_ref, k_ref, v_ref, wo_ref, out_ref,
             fbuf, t1buf, recv_f, send_f, t1_sends, t1_recvs):

        my = lax.axis_index("i")
        barrier = pltpu.get_barrier_semaphore()

        def barrier_round():
            for r in range(N_DEV):
                @pl.when(my == r)
                def _():
                    for p in PARTNERS[r]:
                        pl.semaphore_signal(barrier, inc=1, device_id=(p,),
                                            device_id_type=_MESH)
                    pl.semaphore_wait(barrier, len(PARTNERS[r]))

        barrier_round()

        def send_chunk(c, children, sent):
            for ci, child in enumerate(children):
                rdma = pltpu.make_async_remote_copy(
                    src_ref=fbuf.at[c],
                    dst_ref=fbuf.at[c],
                    send_sem=send_f.at[ci, c],
                    recv_sem=recv_f.at[c],
                    device_id=(child,),
                    device_id_type=_MESH,
                )
                rdma.start()
                sent.append(rdma)

        def project_chunk(c, wo_bf):
            rows = (c % CPB) * FW
            return jnp.dot(fbuf[c], wo_bf[rows:rows + FW, :],
                           preferred_element_type=jnp.float32)

        def t1_rdma(b, src_rank):
            return pltpu.make_async_remote_copy(
                src_ref=t1buf.at[b], dst_ref=t1buf.at[b],
                send_sem=t1_sends.at[b], recv_sem=t1_recvs.at[b],
                device_id=(1 - src_rank,), device_id_type=_MESH,
            )

        @pl.when(my == 1)
        def _():
            wq_bf = wq_ref[...].astype(_BF)
            pen = _penalty(NQ1, NK1, QLO, SQ)
            t1s = []
            for b in range(B):
                q_sub = jnp.dot(x_ref[b, QLO:SQ, :].astype(_BF), wq_bf,
                                preferred_element_type=jnp.float32)
                for h in range(HQ):
                    q_bh = q_sub[:, h * DH:(h + 1) * DH].astype(_BF)
                    s_mat = jnp.dot(q_bh, k_ref[b, h, :, 0:NK1].astype(_BF),
                                    preferred_element_type=jnp.float32)
                    w = jnp.exp(s_mat * 0.125 + pen)
                    ssum = jnp.sum(w, axis=1, keepdims=True)
                    ctx = jnp.dot(w.astype(_BF),
                                  v_ref[b, h, 0:NK1, :].astype(_BF),
                                  preferred_element_type=jnp.float32)
                    t1buf[b, :, h * DH:(h + 1) * DH] = ctx
                    t1buf[b, :, T1C + h:T1C + h + 1] = ssum
                rdma = t1_rdma(b, src_rank=1)
                rdma.start()
                t1s.append(rdma)
            for rdma in t1s:
                rdma.wait_send()

        @pl.when(my == 0)
        def _():
            wq_bf = wq_ref[...].astype(_BF)
            pen = _penalty(SQ, SQ, 0, 0)
            sent = []
            for b in range(B):
                q_b = jnp.dot(x_ref[b].astype(_BF), wq_bf,
                              preferred_element_type=jnp.float32)
                t1_rdma(b, src_rank=0).wait_recv()
                for h in range(HQ):
                    i = b * HQ + h
                    q_bh = q_b[:, h * DH:(h + 1) * DH].astype(_BF)
                    s_mat = jnp.dot(q_bh, k_ref[b, h].astype(_BF),
                                    preferred_element_type=jnp.float32)
                    w = jnp.exp(s_mat * 0.125 + pen)
                    ssum = jnp.sum(w, axis=1, keepdims=True)
                    ctx = jnp.dot(w.astype(_BF), v_ref[b, h].astype(_BF),
                                  preferred_element_type=jnp.float32)
                    c, half = i // SPC, (i % SPC) * DH
                    fbuf[c, 0:QLO, half:half + DH] = (
                        ctx[0:QLO] * (1.0 / ssum[0:QLO])).astype(_BF)
                    s1 = t1buf[b, :, T1C + h:T1C + h + 1]
                    c1 = t1buf[b, :, h * DH:(h + 1) * DH]
                    sb = ssum[QLO:SQ] + s1
                    cb = ctx[QLO:SQ] + c1
                    fbuf[c, QLO:SQ, half:half + DH] = (
                        cb * (1.0 / sb)).astype(_BF)
                    if i % SPC == SPC - 1:
                        send_chunk(c, F_CHILDREN[0], sent)
            wo_bf = wo_ref[...].astype(_BF)
            for b in range(B):
                acc = None
                for c in range(b * CPB, (b + 1) * CPB):
                    p = project_chunk(c, wo_bf)
                    acc = p if acc is None else acc + p
                out_ref[b, :, :] = acc
            for rdma in sent:
                rdma.wait_send()

        for r in range(1, N_DEV):
            @pl.when(my == r)
            def _(r=r):
                wo_bf = wo_ref[...].astype(_BF)
                sent = []
                accs = [None] * B
                for c in range(NCHUNK):
                    recv = pltpu.make_async_remote_copy(
                        src_ref=fbuf.at[c],
                        dst_ref=fbuf.at[c],
                        send_sem=send_f.at[0, c],
                        recv_sem=recv_f.at[c],
                        device_id=(F_PARENT[r],),
                        device_id_type=_MESH,
                    )
                    recv.wait_recv()
                    if F_CHILDREN[r]:
                        send_chunk(c, F_CHILDREN[r], sent)
                    b = c // CPB
                    p = project_chunk(c, wo_bf)
                    accs[b] = p if accs[b] is None else accs[b] + p
                for b in range(B):
                    out_ref[b, :, :] = accs[b]
                for rdma in sent:
                    rdma.wait_send()

        barrier_round()

    return pl.pallas_call(
        body,
        out_shape=jax.ShapeDtypeStruct((B, SQ, D_MODEL), jnp.float32),
        in_specs=[pl.BlockSpec(memory_space=pltpu.VMEM)] * 5,
        out_specs=pl.BlockSpec(memory_space=pltpu.VMEM),
        scratch_shapes=[
            pltpu.VMEM((NCHUNK, SQ, FW), _BF),
            pltpu.VMEM((B, NQ1, T1W), jnp.float32),
            pltpu.SemaphoreType.DMA((NCHUNK,)),
            pltpu.SemaphoreType.DMA((3, NCHUNK)),
            pltpu.SemaphoreType.DMA((B,)),
            pltpu.SemaphoreType.DMA((B,)),
        ],
        compiler_params=pltpu.CompilerParams(collective_id=0),
    )(x, Wq, Kt, Vt, Wo)


# device time: 33616 ns/iter; 1.3183x vs baseline; 1.0855x over previous
import jax
import jax.numpy as jnp
from jax import lax
from jax.experimental import pallas as pl
from jax.experimental.pallas import tpu as pltpu

N_DEV = 8
B, SQ, HQ, DH = 2, 512, 8, 64
WINDOW = 128
D_MODEL = 768
NSLOT = B * HQ
QLO = SQ - WINDOW
NQ1 = SQ - QLO
NK1 = WINDOW
NCHUNK = 8
SPC = NSLOT // NCHUNK
CPB = NCHUNK // B
FW = SPC * DH
T1C = SPC * DH
T1W = T1C + 128

F_PARENT = {1: 0, 3: 0, 4: 0, 2: 1, 5: 1, 6: 2, 7: 4}
F_CHILDREN = {0: [1, 3, 4], 1: [2, 5], 2: [6], 3: [], 4: [7],
              5: [], 6: [], 7: []}
PARTNERS = {
    0: [1, 3, 4],
    1: [0, 2, 5],
    2: [1, 6],
    3: [0],
    4: [0, 7],
    5: [1],
    6: [2],
    7: [4],
}

_BF = jnp.bfloat16
_MESH = pl.DeviceIdType.MESH


def _penalty(nq, nk, q0, k0):
    qi = q0 + lax.broadcasted_iota(jnp.int32, (nq, nk), 0)
    kj = k0 + lax.broadcasted_iota(jnp.int32, (nq, nk), 1)
    return jnp.where(jnp.abs(qi - kj) <= WINDOW, 0.0, -1e9)


def kernel(x, Wq, K_ext, V_ext, Wo):
    Kt = jnp.transpose(K_ext, (0, 2, 3, 1))
    Vt = jnp.transpose(V_ext, (0, 2, 1, 3))

    def body(x_ref, wq_ref, k_ref, v_ref, wo_ref, out_ref,
             fbuf, t1buf, recv_f, send_f, t1_sends, t1_recvs):

        my = lax.axis_index("i")
        barrier = pltpu.get_barrier_semaphore()

        def barrier_round():
            for r in range(N_DEV):
                @pl.when(my == r)
                def _():
                    for p in PARTNERS[r]:
                        pl.semaphore_signal(barrier, inc=1, device_id=(p,),
                                            device_id_type=_MESH)
                    pl.semaphore_wait(barrier, len(PARTNERS[r]))

        barrier_round()

        def send_chunk(c, children, sent):
            for ci, child in enumerate(children):
                rdma = pltpu.make_async_remote_copy(
                    src_ref=fbuf.at[c],
                    dst_ref=fbuf.at[c],
                    send_sem=send_f.at[ci, c],
                    recv_sem=recv_f.at[c],
                    device_id=(child,),
                    device_id_type=_MESH,
                )
                rdma.start()
                sent.append(rdma)

        def project_chunk(c, wo_bf):
            rows = (c % CPB) * FW
            return jnp.dot(fbuf[c], wo_bf[rows:rows + FW, :],
                           preferred_element_type=jnp.float32)

        def t1_rdma(c, src_rank):
            return pltpu.make_async_remote_copy(
                src_ref=t1buf.at[c], dst_ref=t1buf.at[c],
                send_sem=t1_sends.at[c], recv_sem=t1_recvs.at[c],
                device_id=(1 - src_rank,), device_id_type=_MESH,
            )

        @pl.when(my == 1)
        def _():
            wq_bf = wq_ref[...].astype(_BF)
            pen = _penalty(NQ1, NK1, QLO, SQ)
            t1s = []
            for b in range(B):
                q_sub = jnp.dot(x_ref[b, QLO:SQ, :].astype(_BF), wq_bf,
                                preferred_element_type=jnp.float32)
                for h in range(HQ):
                    i = b * HQ + h
                    q_bh = q_sub[:, h * DH:(h + 1) * DH].astype(_BF)
                    s_mat = jnp.dot(q_bh, k_ref[b, h, :, 0:NK1].astype(_BF),
                                    preferred_element_type=jnp.float32)
                    w = jnp.exp(s_mat * 0.125 + pen)
                    ssum = jnp.sum(w, axis=1, keepdims=True)
                    ctx = jnp.dot(w.astype(_BF),
                                  v_ref[b, h, 0:NK1, :].astype(_BF),
                                  preferred_element_type=jnp.float32)
                    c, half = i // SPC, (i % SPC) * DH
                    t1buf[c, :, half:half + DH] = ctx.astype(_BF)
                    t1buf[c, :, T1C + (i % SPC):T1C + (i % SPC) + 1] = (
                        ssum.astype(_BF))
                    if i % SPC == SPC - 1:
                        rdma = t1_rdma(c, src_rank=1)
                        rdma.start()
                        t1s.append(rdma)
            for rdma in t1s:
                rdma.wait_send()

        @pl.when(my == 0)
        def _():
            wq_bf = wq_ref[...].astype(_BF)
            pen = _penalty(SQ, SQ, 0, 0)
            sent = []
            for b in range(B):
                q_b = jnp.dot(x_ref[b].astype(_BF), wq_bf,
                              preferred_element_type=jnp.float32)
                for h in range(HQ):
                    i = b * HQ + h
                    q_bh = q_b[:, h * DH:(h + 1) * DH].astype(_BF)
                    s_mat = jnp.dot(q_bh, k_ref[b, h].astype(_BF),
                                    preferred_element_type=jnp.float32)
                    w = jnp.exp(s_mat * 0.125 + pen)
                    ssum = jnp.sum(w, axis=1, keepdims=True)
                    ctx = jnp.dot(w.astype(_BF), v_ref[b, h].astype(_BF),
                                  preferred_element_type=jnp.float32)
                    c, half = i // SPC, (i % SPC) * DH
                    fbuf[c, 0:QLO, half:half + DH] = (
                        ctx[0:QLO] * (1.0 / ssum[0:QLO])).astype(_BF)
                    if i % SPC == 0:
                        t1_rdma(c, src_rank=0).wait_recv()
                    s1 = t1buf[c, :, T1C + (i % SPC):T1C + (i % SPC) + 1]
                    c1 = t1buf[c, :, half:half + DH]
                    sb = ssum[QLO:SQ] + s1
                    cb = ctx[QLO:SQ] + c1
                    fbuf[c, QLO:SQ, half:half + DH] = (
                        cb * (1.0 / sb)).astype(_BF)
                    if i % SPC == SPC - 1:
                        send_chunk(c, F_CHILDREN[0], sent)
            wo_bf = wo_ref[...].astype(_BF)
            for b in range(B):
                acc = None
                for c in range(b * CPB, (b + 1) * CPB):
                    p = project_chunk(c, wo_bf)
                    acc = p if acc is None else acc + p
                out_ref[b, :, :] = acc
            for rdma in sent:
                rdma.wait_send()

        for r in range(1, N_DEV):
            @pl.when(my == r)
            def _(r=r):
                wo_bf = wo_ref[...].astype(_BF)
                sent = []
                accs = [None] * B
                for c in range(NCHUNK):
                    recv = pltpu.make_async_remote_copy(
                        src_ref=fbuf.at[c],
                        dst_ref=fbuf.at[c],
                        send_sem=send_f.at[0, c],
                        recv_sem=recv_f.at[c],
                        device_id=(F_PARENT[r],),
                        device_id_type=_MESH,
                    )
                    recv.wait_recv()
                    if F_CHILDREN[r]:
                        send_chunk(c, F_CHILDREN[r], sent)
                    b = c // CPB
                    p = project_chunk(c, wo_bf)
                    accs[b] = p if accs[b] is None else accs[b] + p
                for b in range(B):
                    out_ref[b, :, :] = accs[b]
                for rdma in sent:
                    rdma.wait_send()

        barrier_round()

    return pl.pallas_call(
        body,
        out_shape=jax.ShapeDtypeStruct((B, SQ, D_MODEL), jnp.float32),
        in_specs=[pl.BlockSpec(memory_space=pltpu.VMEM)] * 5,
        out_specs=pl.BlockSpec(memory_space=pltpu.VMEM),
        scratch_shapes=[
            pltpu.VMEM((NCHUNK, SQ, FW), _BF),
            pltpu.VMEM((NCHUNK, NQ1, T1W), _BF),
            pltpu.SemaphoreType.DMA((NCHUNK,)),
            pltpu.SemaphoreType.DMA((3, NCHUNK)),
            pltpu.SemaphoreType.DMA((NCHUNK,)),
            pltpu.SemaphoreType.DMA((NCHUNK,)),
        ],
        compiler_params=pltpu.CompilerParams(collective_id=0),
    )(x, Wq, Kt, Vt, Wo)


# device time: 31732 ns/iter; 1.3966x vs baseline; 1.0594x over previous
import jax
import jax.numpy as jnp
from jax import lax
from jax.experimental import pallas as pl
from jax.experimental.pallas import tpu as pltpu

N_DEV = 8
B, SQ, HQ, DH = 2, 512, 8, 64
WINDOW = 128
D_MODEL = 768
NSLOT = B * HQ
QLO = SQ - WINDOW
NQ1 = SQ - QLO
NK1 = WINDOW
NCHUNK = 8
SPC = NSLOT // NCHUNK
CPB = NCHUNK // B
FW = SPC * DH
T1C = SPC * DH
T1W = T1C + 128

F_PARENT = {1: 0, 3: 0, 4: 0, 2: 1, 5: 1, 6: 2, 7: 4}
F_CHILDREN = {0: [1, 3, 4], 1: [2, 5], 2: [6], 3: [], 4: [7],
              5: [], 6: [], 7: []}
PARTNERS = {
    0: [1, 3, 4],
    1: [0, 2, 5],
    2: [1, 6],
    3: [0],
    4: [0, 7],
    5: [1],
    6: [2],
    7: [4],
}

_BF = jnp.bfloat16
_MESH = pl.DeviceIdType.MESH


def _penalty(nq, nk, q0, k0):
    qi = q0 + lax.broadcasted_iota(jnp.int32, (nq, nk), 0)
    kj = k0 + lax.broadcasted_iota(jnp.int32, (nq, nk), 1)
    return jnp.where(jnp.abs(qi - kj) <= WINDOW, 0.0, -1e9)


def kernel(x, Wq, K_ext, V_ext, Wo):
    K2 = K_ext.reshape(B, SQ, HQ * DH)
    V2 = V_ext.reshape(B, SQ, HQ * DH)

    def body(x_ref, wq_ref, k_ref, v_ref, wo_ref, out_ref,
             fbuf, t1buf, recv_f, send_f, t1_sends, t1_recvs):

        my = lax.axis_index("i")
        barrier = pltpu.get_barrier_semaphore()

        def barrier_round():
            for r in range(N_DEV):
                @pl.when(my == r)
                def _():
                    for p in PARTNERS[r]:
                        pl.semaphore_signal(barrier, inc=1, device_id=(p,),
                                            device_id_type=_MESH)
                    pl.semaphore_wait(barrier, len(PARTNERS[r]))

        barrier_round()

        HR = SQ // 2

        def send_half(c, half, children, sent):
            for ci, child in enumerate(children):
                rdma = pltpu.make_async_remote_copy(
                    src_ref=fbuf.at[c, pl.ds(half * HR, HR)],
                    dst_ref=fbuf.at[c, pl.ds(half * HR, HR)],
                    send_sem=send_f.at[ci, c, half],
                    recv_sem=recv_f.at[c, half],
                    device_id=(child,),
                    device_id_type=_MESH,
                )
                rdma.start()
                sent.append(rdma)

        def send_chunk(c, children, sent):
            for half in range(2):
                send_half(c, half, children, sent)

        def project_chunk(c, wo_bf):
            rows = (c % CPB) * FW
            return jnp.dot(fbuf[c], wo_bf[rows:rows + FW, :],
                           preferred_element_type=jnp.float32)

        def t1_rdma(c, src_rank):
            return pltpu.make_async_remote_copy(
                src_ref=t1buf.at[c], dst_ref=t1buf.at[c],
                send_sem=t1_sends.at[c], recv_sem=t1_recvs.at[c],
                device_id=(1 - src_rank,), device_id_type=_MESH,
            )

        @pl.when(my == 1)
        def _():
            wq_bf = wq_ref[...].astype(_BF)
            pen = _penalty(NQ1, NK1, QLO, SQ)
            t1s = []
            for b in range(B):
                q_sub = jnp.dot(x_ref[b, QLO:SQ, :].astype(_BF), wq_bf,
                                preferred_element_type=jnp.float32)
                for h in range(HQ):
                    i = b * HQ + h
                    q_bh = q_sub[:, h * DH:(h + 1) * DH].astype(_BF)
                    s_mat = lax.dot_general(
                        q_bh, k_ref[b, 0:NK1, h * DH:(h + 1) * DH].astype(_BF),
                        (((1,), (1,)), ((), ())),
                        preferred_element_type=jnp.float32)
                    w = jnp.exp(s_mat * 0.125 + pen)
                    ssum = jnp.sum(w, axis=1, keepdims=True)
                    ctx = jnp.dot(w.astype(_BF),
                                  v_ref[b, 0:NK1,
                                        h * DH:(h + 1) * DH].astype(_BF),
                                  preferred_element_type=jnp.float32)
                    c, half = i // SPC, (i % SPC) * DH
                    t1buf[c, :, half:half + DH] = ctx.astype(_BF)
                    t1buf[c, :, T1C + (i % SPC):T1C + (i % SPC) + 1] = (
                        ssum.astype(_BF))
                    if i % SPC == SPC - 1:
                        rdma = t1_rdma(c, src_rank=1)
                        rdma.start()
                        t1s.append(rdma)
            for rdma in t1s:
                rdma.wait_send()

        @pl.when(my == 0)
        def _():
            wq_bf = wq_ref[...].astype(_BF)
            HB, KB = SQ // 2, 384
            pen_t = _penalty(HB, KB, 0, 0)
            pen_b = _penalty(HB, KB, HB, HB - WINDOW)
            sent = []
            for b in range(B):
                q_b = jnp.dot(x_ref[b].astype(_BF), wq_bf,
                              preferred_element_type=jnp.float32)
                k_all = k_ref[b].astype(_BF)
                v_all = v_ref[b].astype(_BF)
                for h in range(HQ):
                    i = b * HQ + h
                    q_bh = q_b[:, h * DH:(h + 1) * DH].astype(_BF)
                    k_bh = k_all[:, h * DH:(h + 1) * DH]
                    v_bh = v_all[:, h * DH:(h + 1) * DH]
                    s_t = lax.dot_general(
                        q_bh[0:HB], k_bh[0:KB], (((1,), (1,)), ((), ())),
                        preferred_element_type=jnp.float32)
                    w_t = jnp.exp(s_t * 0.125 + pen_t)
                    ss_t = jnp.sum(w_t, axis=1, keepdims=True)
                    ctx_t = jnp.dot(w_t.astype(_BF), v_bh[0:KB],
                                    preferred_element_type=jnp.float32)
                    s_b = lax.dot_general(
                        q_bh[HB:SQ], k_bh[HB - WINDOW:SQ],
                        (((1,), (1,)), ((), ())),
                        preferred_element_type=jnp.float32)
                    w_b = jnp.exp(s_b * 0.125 + pen_b)
                    ss_b = jnp.sum(w_b, axis=1, keepdims=True)
                    ctx_b = jnp.dot(w_b.astype(_BF), v_bh[HB - WINDOW:SQ],
                                    preferred_element_type=jnp.float32)
                    c, half = i // SPC, (i % SPC) * DH
                    nmid = QLO - HB
                    fbuf[c, 0:HB, half:half + DH] = (
                        ctx_t * (1.0 / ss_t)).astype(_BF)
                    fbuf[c, HB:QLO, half:half + DH] = (
                        ctx_b[0:nmid] * (1.0 / ss_b[0:nmid])).astype(_BF)
                    if i % SPC == 0:
                        t1_rdma(c, src_rank=0).wait_recv()
                    s1 = t1buf[c, :, T1C + (i % SPC):T1C + (i % SPC) + 1]
                    c1 = t1buf[c, :, half:half + DH]
                    sb = ss_b[nmid:HB] + s1
                    cb = ctx_b[nmid:HB] + c1
                    fbuf[c, QLO:SQ, half:half + DH] = (
                        cb * (1.0 / sb)).astype(_BF)
                    if i % SPC == SPC - 1:
                        send_chunk(c, F_CHILDREN[0], sent)
            wo_bf = wo_ref[...].astype(_BF)
            for b in range(B):
                acc = None
                for c in range(b * CPB, (b + 1) * CPB):
                    p = project_chunk(c, wo_bf)
                    acc = p if acc is None else acc + p
                out_ref[b, :, :] = acc
            for rdma in sent:
                rdma.wait_send()

        for r in range(1, N_DEV):
            @pl.when(my == r)
            def _(r=r):
                wo_bf = wo_ref[...].astype(_BF)
                sent = []
                accs = [None] * B
                for c in range(NCHUNK):
                    for half in range(2):
                        recv = pltpu.make_async_remote_copy(
                            src_ref=fbuf.at[c, pl.ds(half * HR, HR)],
                            dst_ref=fbuf.at[c, pl.ds(half * HR, HR)],
                            send_sem=send_f.at[0, c, half],
                            recv_sem=recv_f.at[c, half],
                            device_id=(F_PARENT[r],),
                            device_id_type=_MESH,
                        )
                        recv.wait_recv()
                        if F_CHILDREN[r]:
                            send_half(c, half, F_CHILDREN[r], sent)
                    b = c // CPB
                    p = project_chunk(c, wo_bf)
                    accs[b] = p if accs[b] is None else accs[b] + p
                for b in range(B):
                    out_ref[b, :, :] = accs[b]
                for rdma in sent:
                    rdma.wait_send()

        barrier_round()

    return pl.pallas_call(
        body,
        out_shape=jax.ShapeDtypeStruct((B, SQ, D_MODEL), jnp.float32),
        in_specs=[pl.BlockSpec(memory_space=pltpu.VMEM)] * 5,
        out_specs=pl.BlockSpec(memory_space=pltpu.VMEM),
        scratch_shapes=[
            pltpu.VMEM((NCHUNK, SQ, FW), _BF),
            pltpu.VMEM((NCHUNK, NQ1, T1W), _BF),
            pltpu.SemaphoreType.DMA((NCHUNK, 2)),
            pltpu.SemaphoreType.DMA((3, NCHUNK, 2)),
            pltpu.SemaphoreType.DMA((NCHUNK,)),
            pltpu.SemaphoreType.DMA((NCHUNK,)),
        ],
        compiler_params=pltpu.CompilerParams(collective_id=0),
    )(x, Wq, K2, V2, Wo)
